# Initial kernel scaffold; baseline (speedup 1.0000x reference)
import jax
import jax.numpy as jnp
from jax.experimental import pallas as pl


def kernel(x, view_dir, tables, W0, b0, W1, b1, W2, b2):
    raise NotImplementedError("write your pallas kernel here")


# speedup vs baseline: 7.3006x; 7.3006x over previous
import functools
import numpy as np
import jax
import jax.numpy as jnp
from jax import lax
from jax.experimental import pallas as pl
from jax.experimental.pallas import tpu as pltpu
from jax.experimental.pallas import tpu_sc as plsc

_LEVELS = 16
_FEATURES = 2
_HASHMAP = 524288
_BASE_RES = 16
_SCALE = 1.3819
_N = 524288
_HIDDEN = 64

_NC, _NS = 2, 16
_NW = _NC * _NS
_CHUNK = _N // _NW
_SUB = 128
_NSUB = _CHUNK // _SUB

_C0 = np.int32(73856093)
_C1 = np.int32(19349663)
_C2 = np.int32(83492791)
_RES = [np.float32(int(_BASE_RES * _SCALE ** l)) for l in range(_LEVELS)]


def _sc_embed_body(x_hbm, tbl_hbm, out_hbm, xv, idxv, lowv, rows, outv, sem):
    wid = lax.axis_index("c") * _NS + lax.axis_index("s")
    iota = lax.iota(jnp.int32, 16)

    @pl.loop(0, _NSUB)
    def _sub(g):
        base = wid * _CHUNK + g * _SUB
        pltpu.sync_copy(x_hbm.at[pl.ds(base * 3, _SUB * 3)], xv)

        @pl.loop(0, _SUB // 16)
        def _hash(gg):
            p0 = gg * 16
            pidx3 = (p0 + iota) * 3
            xx = plsc.load_gather(xv, [pidx3])
            yy = plsc.load_gather(xv, [pidx3 + 1])
            zz = plsc.load_gather(xv, [pidx3 + 2])
            for l in range(_LEVELS):
                px = (xx * _RES[l]).astype(jnp.int32)
                py = (yy * _RES[l]).astype(jnp.int32)
                pz = (zz * _RES[l]).astype(jnp.int32)
                h = (px * _C0) ^ (py * _C1) ^ (pz * _C2)
                slot = (h & np.int32(_HASHMAP - 1)) | np.int32(l * _HASHMAP)
                idxv[l, pl.ds(p0, 16)] = lax.shift_right_logical(
                    slot, np.int32(2))
                lowv[l, pl.ds(p0, 16)] = (slot & np.int32(3)) * np.int32(2)

        copies = [
            pltpu.async_copy(tbl_hbm.at[idxv.at[l]], rows.at[l], sem)
            for l in range(_LEVELS)
        ]
        for c in copies:
            c.wait()

        @pl.loop(0, _SUB // 16)
        def _transpose(gg):
            p0 = gg * 16
            p_idx = p0 + iota
            for l in range(_LEVELS):
                low = lowv[l, pl.ds(p0, 16)]
                lsplat = jnp.full((16,), l, jnp.int32)
                for f in range(_FEATURES):
                    vals = plsc.load_gather(rows, [lsplat, p_idx, low + f])
                    outv[2 * l + f, pl.ds(p0, 16)] = vals

        pltpu.sync_copy(outv, out_hbm.at[:, pl.ds(base, _SUB)])


@functools.cache
def _get_sc_embed():
    return pl.kernel(
        _sc_embed_body,
        out_type=jax.ShapeDtypeStruct((2 * _LEVELS, _N), jnp.float32),
        mesh=plsc.VectorSubcoreMesh(
            core_axis_name="c", subcore_axis_name="s",
            num_cores=_NC, num_subcores=_NS),
        scratch_types=[
            pltpu.VMEM((_SUB * 3,), jnp.float32),
            pltpu.VMEM((_LEVELS, _SUB), jnp.int32),
            pltpu.VMEM((_LEVELS, _SUB), jnp.int32),
            pltpu.VMEM((_LEVELS, _SUB, 8), jnp.float32),
            pltpu.VMEM((2 * _LEVELS, _SUB), jnp.float32),
            pltpu.SemaphoreType.DMA,
        ],
        compiler_params=pltpu.CompilerParams(
            needs_layout_passes=False, use_tc_tiling_on_sc=False),
    )


_BLK = 4096


def _mlp_body(ht_ref, vd_ref, w0a_ref, w0b_ref, b0_ref, w1_ref, b1_ref,
              w2_ref, b2_ref, rgb_ref, sig_ref):
    z0 = jax.lax.dot_general(
        ht_ref[...], w0a_ref[...], (((0,), (0,)), ((), ())),
        preferred_element_type=jnp.float32)
    z0 = z0 + jnp.dot(vd_ref[...], w0b_ref[...],
                      preferred_element_type=jnp.float32)
    h1 = jnp.maximum(z0 + b0_ref[...], 0.0)
    z1 = jnp.dot(h1, w1_ref[...], preferred_element_type=jnp.float32)
    h2 = jnp.maximum(z1 + b1_ref[...], 0.0)
    o = jnp.dot(h2, w2_ref[...], preferred_element_type=jnp.float32)
    o = o + b2_ref[...]
    rgb_ref[...] = 1.0 / (1.0 + jnp.exp(-o[:, :3]))
    sig_ref[...] = jnp.maximum(o[:, 3:4], 0.0)


def _mlp(ht, vd, w0a, w0b, b0, w1, b1, w2, b2):
    nblk = _N // _BLK
    full = lambda i: (0, 0)
    return pl.pallas_call(
        _mlp_body,
        grid=(nblk,),
        in_specs=[
            pl.BlockSpec((2 * _LEVELS, _BLK), lambda i: (0, i)),
            pl.BlockSpec((_BLK, 3), lambda i: (i, 0)),
            pl.BlockSpec((2 * _LEVELS, _HIDDEN), full),
            pl.BlockSpec((3, _HIDDEN), full),
            pl.BlockSpec((1, _HIDDEN), full),
            pl.BlockSpec((_HIDDEN, _HIDDEN), full),
            pl.BlockSpec((1, _HIDDEN), full),
            pl.BlockSpec((_HIDDEN, 4), full),
            pl.BlockSpec((1, 4), full),
        ],
        out_specs=[
            pl.BlockSpec((_BLK, 3), lambda i: (i, 0)),
            pl.BlockSpec((_BLK, 1), lambda i: (i, 0)),
        ],
        out_shape=[
            jax.ShapeDtypeStruct((_N, 3), jnp.float32),
            jax.ShapeDtypeStruct((_N, 1), jnp.float32),
        ],
    )(ht, vd, w0a, w0b, b0, w1, b1, w2, b2)


@jax.jit
def kernel(x, view_dir, tables, W0, b0, W1, b1, W2, b2):
    tbl = tables.reshape(_LEVELS * _HASHMAP // 4, 8)
    ht = _get_sc_embed()(x.reshape(-1), tbl)
    rgb, sigma = _mlp(ht, view_dir,
                      W0[:2 * _LEVELS], W0[2 * _LEVELS:], b0.reshape(1, -1),
                      W1, b1.reshape(1, -1), W2, b2.reshape(1, -1))
    return (rgb, sigma)


# speedup vs baseline: 36.7440x; 5.0330x over previous
import functools
import numpy as np
import jax
import jax.numpy as jnp
from jax import lax
from jax.experimental import pallas as pl
from jax.experimental.pallas import tpu as pltpu
from jax.experimental.pallas import tpu_sc as plsc

_LEVELS = 16
_FEATURES = 2
_HASHMAP = 524288
_BASE_RES = 16
_SCALE = 1.3819
_N = 524288
_HIDDEN = 64

_NC, _NS = 2, 16
_NW = _NC * _NS
_CHUNK = _N // _NW
_SUB = 128
_NSUB = _CHUNK // _SUB

_C0 = np.int32(73856093)
_C1 = np.int32(19349663)
_C2 = np.int32(83492791)
_RES = [np.float32(int(_BASE_RES * _SCALE ** l)) for l in range(_LEVELS)]


def _sc_embed_body(x_hbm, tbl_hbm, out_hbm, xv, idx0, idx1, lowv, rows, outv,
                   sem):
    wid = lax.axis_index("c") * _NS + lax.axis_index("s")
    iota = lax.iota(jnp.int32, 16)

    @pl.loop(0, _NSUB)
    def _sub(g):
        base = wid * _CHUNK + g * _SUB
        pltpu.sync_copy(x_hbm.at[pl.ds(base * 3, _SUB * 3)], xv)

        @pl.loop(0, _SUB // 16)
        def _hash(gg):
            p0 = gg * 16
            pidx3 = (p0 + iota) * 3
            xx = plsc.load_gather(xv, [pidx3])
            yy = plsc.load_gather(xv, [pidx3 + 1])
            zz = plsc.load_gather(xv, [pidx3 + 2])
            for l in range(_LEVELS):
                px = (xx * _RES[l]).astype(jnp.int32)
                py = (yy * _RES[l]).astype(jnp.int32)
                pz = (zz * _RES[l]).astype(jnp.int32)
                h = (px * _C0) ^ (py * _C1) ^ (pz * _C2)
                slot = h & np.int32(_HASHMAP - 1)
                r0 = (lax.shift_right_logical(slot & np.int32(0x7FF80),
                                              np.int32(2))
                      | (lax.shift_right_logical(slot, np.int32(3))
                         & np.int32(15))
                      | np.int32(l * 131072))
                idx0[l, pl.ds(p0, 16)] = r0
                idx1[l, pl.ds(p0, 16)] = r0 | np.int32(16)
                lowv[l, pl.ds(p0, 16)] = slot & np.int32(7)

        copies = []
        for l in range(_LEVELS):
            copies.append(pltpu.async_copy(
                tbl_hbm.at[idx0.at[l]], rows.at[l, 0], sem))
            copies.append(pltpu.async_copy(
                tbl_hbm.at[idx1.at[l]], rows.at[l, 1], sem))
        for c in copies:
            c.wait()

        @pl.loop(0, _SUB // 16)
        def _transpose(gg):
            p0 = gg * 16
            p_idx = p0 + iota
            for l in range(_LEVELS):
                low = lowv[l, pl.ds(p0, 16)]
                lsplat = jnp.full((16,), l, jnp.int32)
                for f in range(_FEATURES):
                    fsplat = jnp.full((16,), f, jnp.int32)
                    vals = plsc.load_gather(
                        rows, [lsplat, fsplat, p_idx, low])
                    outv[2 * l + f, pl.ds(p0, 16)] = vals

        pltpu.sync_copy(outv, out_hbm.at[:, pl.ds(base, _SUB)])


@functools.cache
def _get_sc_embed():
    return pl.kernel(
        _sc_embed_body,
        out_type=jax.ShapeDtypeStruct((2 * _LEVELS, _N), jnp.float32),
        mesh=plsc.VectorSubcoreMesh(
            core_axis_name="c", subcore_axis_name="s",
            num_cores=_NC, num_subcores=_NS),
        scratch_types=[
            pltpu.VMEM((_SUB * 3,), jnp.float32),
            pltpu.VMEM((_LEVELS, _SUB), jnp.int32),
            pltpu.VMEM((_LEVELS, _SUB), jnp.int32),
            pltpu.VMEM((_LEVELS, _SUB), jnp.int32),
            pltpu.VMEM((_LEVELS, _FEATURES, _SUB, 8), jnp.float32),
            pltpu.VMEM((2 * _LEVELS, _SUB), jnp.float32),
            pltpu.SemaphoreType.DMA,
        ],
        compiler_params=pltpu.CompilerParams(
            needs_layout_passes=False, use_tc_tiling_on_sc=False),
    )


_BLK = 4096


def _mlp_body(ht_ref, vd_ref, w0a_ref, w0b_ref, b0_ref, w1_ref, b1_ref,
              w2_ref, b2_ref, rgb_ref, sig_ref):
    z0 = jax.lax.dot_general(
        ht_ref[...], w0a_ref[...], (((0,), (0,)), ((), ())),
        preferred_element_type=jnp.float32)
    z0 = z0 + jnp.dot(vd_ref[...], w0b_ref[...],
                      preferred_element_type=jnp.float32)
    h1 = jnp.maximum(z0 + b0_ref[...], 0.0)
    z1 = jnp.dot(h1, w1_ref[...], preferred_element_type=jnp.float32)
    h2 = jnp.maximum(z1 + b1_ref[...], 0.0)
    o = jnp.dot(h2, w2_ref[...], preferred_element_type=jnp.float32)
    o = o + b2_ref[...]
    rgb_ref[...] = 1.0 / (1.0 + jnp.exp(-o[:, :3]))
    sig_ref[...] = jnp.maximum(o[:, 3:4], 0.0)


def _mlp(ht, vd, w0a, w0b, b0, w1, b1, w2, b2):
    nblk = _N // _BLK
    full = lambda i: (0, 0)
    return pl.pallas_call(
        _mlp_body,
        grid=(nblk,),
        in_specs=[
            pl.BlockSpec((2 * _LEVELS, _BLK), lambda i: (0, i)),
            pl.BlockSpec((_BLK, 3), lambda i: (i, 0)),
            pl.BlockSpec((2 * _LEVELS, _HIDDEN), full),
            pl.BlockSpec((3, _HIDDEN), full),
            pl.BlockSpec((1, _HIDDEN), full),
            pl.BlockSpec((_HIDDEN, _HIDDEN), full),
            pl.BlockSpec((1, _HIDDEN), full),
            pl.BlockSpec((_HIDDEN, 4), full),
            pl.BlockSpec((1, 4), full),
        ],
        out_specs=[
            pl.BlockSpec((_BLK, 3), lambda i: (i, 0)),
            pl.BlockSpec((_BLK, 1), lambda i: (i, 0)),
        ],
        out_shape=[
            jax.ShapeDtypeStruct((_N, 3), jnp.float32),
            jax.ShapeDtypeStruct((_N, 1), jnp.float32),
        ],
    )(ht, vd, w0a, w0b, b0, w1, b1, w2, b2)


@jax.jit
def kernel(x, view_dir, tables, W0, b0, W1, b1, W2, b2):
    tbl = (tables.reshape(_LEVELS, _HASHMAP // 128, 128, _FEATURES)
           .transpose(0, 1, 3, 2)
           .reshape(_LEVELS * _HASHMAP * _FEATURES // 8, 8))
    ht = _get_sc_embed()(x.reshape(-1), tbl)
    rgb, sigma = _mlp(ht, view_dir,
                      W0[:2 * _LEVELS], W0[2 * _LEVELS:], b0.reshape(1, -1),
                      W1, b1.reshape(1, -1), W2, b2.reshape(1, -1))
    return (rgb, sigma)


# speedup vs baseline: 54.8652x; 1.4932x over previous
import functools
import numpy as np
import jax
import jax.numpy as jnp
from jax import lax
from jax.experimental import pallas as pl
from jax.experimental.pallas import tpu as pltpu
from jax.experimental.pallas import tpu_sc as plsc

_LEVELS = 16
_FEATURES = 2
_HASHMAP = 524288
_BASE_RES = 16
_SCALE = 1.3819
_N = 524288
_HIDDEN = 64

_NC, _NS = 2, 16
_NW = _NC * _NS
_CHUNK = _N // _NW
_SUB = 128
_NSUB = _CHUNK // _SUB

_C0 = np.int32(73856093)
_C1 = np.int32(19349663)
_C2 = np.int32(83492791)
_RES = [np.float32(int(_BASE_RES * _SCALE ** l)) for l in range(_LEVELS)]


def _sc_embed_body(x_hbm, tbl_hbm, out_hbm, xv, idx0, idx1, lowv, rows, outv,
                   sem):
    wid = lax.axis_index("c") * _NS + lax.axis_index("s")
    iota = lax.iota(jnp.int32, 16)

    @pl.loop(0, _NSUB)
    def _sub(g):
        pb = wid * _NSUB + g
        pltpu.sync_copy(x_hbm.at[:, pb], xv)

        @pl.loop(0, _SUB // 16)
        def _hash(gg):
            p0 = gg * 16
            xx = xv[0, pl.ds(p0, 16)]
            yy = xv[1, pl.ds(p0, 16)]
            zz = xv[2, pl.ds(p0, 16)]
            for l in range(_LEVELS):
                px = (xx * _RES[l]).astype(jnp.int32)
                py = (yy * _RES[l]).astype(jnp.int32)
                pz = (zz * _RES[l]).astype(jnp.int32)
                h = (px * _C0) ^ (py * _C1) ^ (pz * _C2)
                slot = h & np.int32(_HASHMAP - 1)
                r0 = (lax.shift_right_logical(slot & np.int32(0x7FF80),
                                              np.int32(2))
                      | (lax.shift_right_logical(slot, np.int32(3))
                         & np.int32(15))
                      | np.int32(l * 131072))
                idx0[l, pl.ds(p0, 16)] = r0
                idx1[l, pl.ds(p0, 16)] = r0 | np.int32(16)
                lowv[l, pl.ds(p0, 16)] = slot & np.int32(7)

        copies = []
        for l in range(_LEVELS):
            copies.append(pltpu.async_copy(
                tbl_hbm.at[idx0.at[l]], rows.at[l, 0], sem))
            copies.append(pltpu.async_copy(
                tbl_hbm.at[idx1.at[l]], rows.at[l, 1], sem))
        for c in copies:
            c.wait()

        @pl.loop(0, _SUB // 16)
        def _transpose(gg):
            p0 = gg * 16
            p_idx = p0 + iota
            for l in range(_LEVELS):
                low = lowv[l, pl.ds(p0, 16)]
                lsplat = jnp.full((16,), l, jnp.int32)
                for f in range(_FEATURES):
                    fsplat = jnp.full((16,), f, jnp.int32)
                    vals = plsc.load_gather(
                        rows, [lsplat, fsplat, p_idx, low])
                    c = 2 * l + f
                    outv[c // 8, c % 8, pl.ds(p0, 16)] = vals

        pltpu.sync_copy(outv, out_hbm.at[:, pb])


@functools.cache
def _get_sc_embed():
    return pl.kernel(
        _sc_embed_body,
        out_type=jax.ShapeDtypeStruct((4, _N // _SUB, 8, _SUB), jnp.float32),
        mesh=plsc.VectorSubcoreMesh(
            core_axis_name="c", subcore_axis_name="s",
            num_cores=_NC, num_subcores=_NS),
        scratch_types=[
            pltpu.VMEM((3, _SUB), jnp.float32),
            pltpu.VMEM((_LEVELS, _SUB), jnp.int32),
            pltpu.VMEM((_LEVELS, _SUB), jnp.int32),
            pltpu.VMEM((_LEVELS, _SUB), jnp.int32),
            pltpu.VMEM((_LEVELS, _FEATURES, _SUB, 8), jnp.float32),
            pltpu.VMEM((4, 8, _SUB), jnp.float32),
            pltpu.SemaphoreType.DMA,
        ],
        compiler_params=pltpu.CompilerParams(
            needs_layout_passes=False, use_tc_tiling_on_sc=False),
    )


_BLK = 4096


def _prep_body(x_ref, xt_ref):
    eye = jnp.eye(3, dtype=jnp.float32)
    xt = jax.lax.dot_general(eye, x_ref[...], (((1,), (1,)), ((), ())),
                             precision=jax.lax.Precision.HIGHEST,
                             preferred_element_type=jnp.float32)
    for j in range(_BLK // _SUB):
        xt_ref[:, j] = xt[:, j * _SUB:(j + 1) * _SUB]


def _prep(x):
    return pl.pallas_call(
        _prep_body,
        grid=(_N // _BLK,),
        in_specs=[pl.BlockSpec((_BLK, 3), lambda i: (i, 0))],
        out_specs=pl.BlockSpec((3, _BLK // _SUB, _SUB), lambda i: (0, i, 0)),
        out_shape=jax.ShapeDtypeStruct((3, _N // _SUB, _SUB), jnp.float32),
    )(x)


def _mlp_body(ht_ref, vd_ref, w0a_ref, w0b_ref, b0_ref, w1_ref, b1_ref,
              w2_ref, b2_ref, rgbt_ref, sig_ref):
    dg = functools.partial(jax.lax.dot_general,
                           preferred_element_type=jnp.float32)
    z0v = dg(w0b_ref[...], vd_ref[...], (((0,), (1,)), ((), ())))
    z0 = dg(w0a_ref[...], ht_ref[...], (((0,), (0,)), ((), ())))
    h1 = jnp.maximum(z0 + z0v + b0_ref[...], 0.0)
    z1 = dg(w1_ref[...], h1, (((0,), (0,)), ((), ())))
    h2 = jnp.maximum(z1 + b1_ref[...], 0.0)
    ot = dg(w2_ref[...], h2, (((0,), (0,)), ((), ())))
    ot = ot + b2_ref[...]
    rgbt_ref[...] = 1.0 / (1.0 + jnp.exp(-ot[:3, :]))
    sig_ref[...] = jnp.maximum(ot[3:4, :], 0.0)


def _mlp(ht, vd, w0a, w0b, b0, w1, b1, w2, b2):
    nblk = _N // _BLK
    full = lambda i: (0, 0)
    return pl.pallas_call(
        _mlp_body,
        grid=(nblk,),
        in_specs=[
            pl.BlockSpec((2 * _LEVELS, _BLK), lambda i: (0, i)),
            pl.BlockSpec((_BLK, 3), lambda i: (i, 0)),
            pl.BlockSpec((2 * _LEVELS, _HIDDEN), full),
            pl.BlockSpec((3, _HIDDEN), full),
            pl.BlockSpec((_HIDDEN, 1), full),
            pl.BlockSpec((_HIDDEN, _HIDDEN), full),
            pl.BlockSpec((_HIDDEN, 1), full),
            pl.BlockSpec((_HIDDEN, 4), full),
            pl.BlockSpec((4, 1), full),
        ],
        out_specs=[
            pl.BlockSpec((3, _BLK), lambda i: (0, i)),
            pl.BlockSpec((1, _BLK), lambda i: (0, i)),
        ],
        out_shape=[
            jax.ShapeDtypeStruct((3, _N), jnp.float32),
            jax.ShapeDtypeStruct((1, _N), jnp.float32),
        ],
    )(ht, vd, w0a, w0b, b0, w1, b1, w2, b2)


@jax.jit
def kernel(x, view_dir, tables, W0, b0, W1, b1, W2, b2):
    tbl = (tables.reshape(_LEVELS, _HASHMAP // 128, 128, _FEATURES)
           .transpose(0, 1, 3, 2)
           .reshape(_LEVELS * _HASHMAP * _FEATURES // 8, 8))
    xt = _prep(x)
    h4 = _get_sc_embed()(xt, tbl)
    ht = h4.transpose(0, 2, 1, 3).reshape(2 * _LEVELS, _N)
    rgbt, sigt = _mlp(ht, view_dir,
                      W0[:2 * _LEVELS], W0[2 * _LEVELS:], b0.reshape(-1, 1),
                      W1, b1.reshape(-1, 1), W2, b2.reshape(-1, 1))
    return (rgbt.T, sigt.reshape(_N, 1))


# speedup vs baseline: 74.8146x; 1.3636x over previous
import functools
import numpy as np
import jax
import jax.numpy as jnp
from jax import lax
from jax.experimental import pallas as pl
from jax.experimental.pallas import tpu as pltpu
from jax.experimental.pallas import tpu_sc as plsc

_LEVELS = 16
_FEATURES = 2
_HASHMAP = 524288
_BASE_RES = 16
_SCALE = 1.3819
_N = 524288
_HIDDEN = 64

_NC, _NS = 2, 16
_NW = _NC * _NS
_CHUNK = _N // _NW
_SUB = 128
_NSUB = _CHUNK // _SUB

_C0 = np.int32(73856093)
_C1 = np.int32(19349663)
_C2 = np.int32(83492791)
_RES = [np.float32(int(_BASE_RES * _SCALE ** l)) for l in range(_LEVELS)]


def _sc_embed_body(x_hbm, tbl_hbm, out_hbm, xv, idx0, idx1, lowv, rows, outv,
                   gsem0, gsem1, osem0, osem1):
    wid = lax.axis_index("c") * _NS + lax.axis_index("s")
    iota = lax.iota(jnp.int32, 16)
    gsems = (gsem0, gsem1)
    osems = (osem0, osem1)

    def prep(g, b):
        pb = wid * _NSUB + g
        pltpu.sync_copy(x_hbm.at[:, pb], xv.at[b])

        @pl.loop(0, _SUB // 16)
        def _hash(gg):
            p0 = gg * 16
            xx = xv[b, 0, pl.ds(p0, 16)]
            yy = xv[b, 1, pl.ds(p0, 16)]
            zz = xv[b, 2, pl.ds(p0, 16)]
            for l in range(_LEVELS):
                px = (xx * _RES[l]).astype(jnp.int32)
                py = (yy * _RES[l]).astype(jnp.int32)
                pz = (zz * _RES[l]).astype(jnp.int32)
                h = (px * _C0) ^ (py * _C1) ^ (pz * _C2)
                slot = h & np.int32(_HASHMAP - 1)
                r0 = (lax.shift_right_logical(slot & np.int32(0x7FF80),
                                              np.int32(2))
                      | (lax.shift_right_logical(slot, np.int32(3))
                         & np.int32(15))
                      | np.int32(l * 131072))
                idx0[b, l, pl.ds(p0, 16)] = r0
                idx1[b, l, pl.ds(p0, 16)] = r0 | np.int32(16)
                lowv[b, l, pl.ds(p0, 16)] = slot & np.int32(7)

        for l in range(_LEVELS):
            pltpu.async_copy(
                tbl_hbm.at[idx0.at[b, l]], rows.at[b, l, 0], gsems[b])
            pltpu.async_copy(
                tbl_hbm.at[idx1.at[b, l]], rows.at[b, l, 1], gsems[b])

    def drain(g, b):
        pb = wid * _NSUB + g
        for l in range(_LEVELS):
            pltpu.make_async_copy(
                tbl_hbm.at[idx0.at[b, l]], rows.at[b, l, 0], gsems[b]).wait()
            pltpu.make_async_copy(
                tbl_hbm.at[idx1.at[b, l]], rows.at[b, l, 1], gsems[b]).wait()

        @pl.when(g >= 2)
        def _():
            pltpu.make_async_copy(
                outv.at[b], out_hbm.at[:, pb], osems[b]).wait()

        @pl.loop(0, _SUB // 16)
        def _transpose(gg):
            p0 = gg * 16
            p_idx = p0 + iota
            for l in range(_LEVELS):
                low = lowv[b, l, pl.ds(p0, 16)]
                lsplat = jnp.full((16,), l, jnp.int32)
                for f in range(_FEATURES):
                    fsplat = jnp.full((16,), f, jnp.int32)
                    vals = plsc.load_gather(
                        rows.at[b], [lsplat, fsplat, p_idx, low])
                    c = 2 * l + f
                    outv[b, c // 8, c % 8, pl.ds(p0, 16)] = vals

        pltpu.async_copy(outv.at[b], out_hbm.at[:, pb], osems[b])

    prep(0, 0)

    @pl.loop(0, _NSUB // 2)
    def _sub(gh):
        g0 = 2 * gh
        prep(g0 + 1, 1)
        drain(g0, 0)

        @pl.when(g0 + 2 < _NSUB)
        def _():
            prep(g0 + 2, 0)

        drain(g0 + 1, 1)

    last = wid * _NSUB + _NSUB - 1
    pltpu.make_async_copy(outv.at[0], out_hbm.at[:, last - 1], osems[0]).wait()
    pltpu.make_async_copy(outv.at[1], out_hbm.at[:, last], osems[1]).wait()


@functools.cache
def _get_sc_embed():
    return pl.kernel(
        _sc_embed_body,
        out_type=jax.ShapeDtypeStruct((4, _N // _SUB, 8, _SUB), jnp.float32),
        mesh=plsc.VectorSubcoreMesh(
            core_axis_name="c", subcore_axis_name="s",
            num_cores=_NC, num_subcores=_NS),
        scratch_types=[
            pltpu.VMEM((2, 3, _SUB), jnp.float32),
            pltpu.VMEM((2, _LEVELS, _SUB), jnp.int32),
            pltpu.VMEM((2, _LEVELS, _SUB), jnp.int32),
            pltpu.VMEM((2, _LEVELS, _SUB), jnp.int32),
            pltpu.VMEM((2, _LEVELS, _FEATURES, _SUB, 8), jnp.float32),
            pltpu.VMEM((2, 4, 8, _SUB), jnp.float32),
            pltpu.SemaphoreType.DMA,
            pltpu.SemaphoreType.DMA,
            pltpu.SemaphoreType.DMA,
            pltpu.SemaphoreType.DMA,
        ],
        compiler_params=pltpu.CompilerParams(
            needs_layout_passes=False, use_tc_tiling_on_sc=False),
    )


_BLK = 4096


def _prep_body(x_ref, xt_ref):
    eye = jnp.eye(3, dtype=jnp.float32)
    xt = jax.lax.dot_general(eye, x_ref[...], (((1,), (1,)), ((), ())),
                             precision=jax.lax.Precision.HIGHEST,
                             preferred_element_type=jnp.float32)
    for j in range(_BLK // _SUB):
        xt_ref[:, j] = xt[:, j * _SUB:(j + 1) * _SUB]


def _prep(x):
    return pl.pallas_call(
        _prep_body,
        grid=(_N // _BLK,),
        in_specs=[pl.BlockSpec((_BLK, 3), lambda i: (i, 0))],
        out_specs=pl.BlockSpec((3, _BLK // _SUB, _SUB), lambda i: (0, i, 0)),
        out_shape=jax.ShapeDtypeStruct((3, _N // _SUB, _SUB), jnp.float32),
    )(x)


def _mlp_body(ht_ref, vd_ref, w0a_ref, w0b_ref, b0_ref, w1_ref, b1_ref,
              w2_ref, b2_ref, rgbt_ref, sig_ref):
    dg = functools.partial(jax.lax.dot_general,
                           preferred_element_type=jnp.float32)
    z0v = dg(w0b_ref[...], vd_ref[...], (((0,), (1,)), ((), ())))
    z0 = dg(w0a_ref[...], ht_ref[...], (((0,), (0,)), ((), ())))
    h1 = jnp.maximum(z0 + z0v + b0_ref[...], 0.0)
    z1 = dg(w1_ref[...], h1, (((0,), (0,)), ((), ())))
    h2 = jnp.maximum(z1 + b1_ref[...], 0.0)
    ot = dg(w2_ref[...], h2, (((0,), (0,)), ((), ())))
    ot = ot + b2_ref[...]
    rgbt_ref[...] = 1.0 / (1.0 + jnp.exp(-ot[:3, :]))
    sig_ref[...] = jnp.maximum(ot[3:4, :], 0.0)


def _mlp(ht, vd, w0a, w0b, b0, w1, b1, w2, b2):
    nblk = _N // _BLK
    full = lambda i: (0, 0)
    return pl.pallas_call(
        _mlp_body,
        grid=(nblk,),
        in_specs=[
            pl.BlockSpec((2 * _LEVELS, _BLK), lambda i: (0, i)),
            pl.BlockSpec((_BLK, 3), lambda i: (i, 0)),
            pl.BlockSpec((2 * _LEVELS, _HIDDEN), full),
            pl.BlockSpec((3, _HIDDEN), full),
            pl.BlockSpec((_HIDDEN, 1), full),
            pl.BlockSpec((_HIDDEN, _HIDDEN), full),
            pl.BlockSpec((_HIDDEN, 1), full),
            pl.BlockSpec((_HIDDEN, 4), full),
            pl.BlockSpec((4, 1), full),
        ],
        out_specs=[
            pl.BlockSpec((3, _BLK), lambda i: (0, i)),
            pl.BlockSpec((1, _BLK), lambda i: (0, i)),
        ],
        out_shape=[
            jax.ShapeDtypeStruct((3, _N), jnp.float32),
            jax.ShapeDtypeStruct((1, _N), jnp.float32),
        ],
    )(ht, vd, w0a, w0b, b0, w1, b1, w2, b2)


@jax.jit
def kernel(x, view_dir, tables, W0, b0, W1, b1, W2, b2):
    tbl = (tables.reshape(_LEVELS, _HASHMAP // 128, 128, _FEATURES)
           .transpose(0, 1, 3, 2)
           .reshape(_LEVELS * _HASHMAP * _FEATURES // 8, 8))
    xt = _prep(x)
    h4 = _get_sc_embed()(xt, tbl)
    ht = h4.transpose(0, 2, 1, 3).reshape(2 * _LEVELS, _N)
    rgbt, sigt = _mlp(ht, view_dir,
                      W0[:2 * _LEVELS], W0[2 * _LEVELS:], b0.reshape(-1, 1),
                      W1, b1.reshape(-1, 1), W2, b2.reshape(-1, 1))
    return (rgbt.T, sigt.reshape(_N, 1))


# speedup vs baseline: 79.8716x; 1.0676x over previous
import functools
import numpy as np
import jax
import jax.numpy as jnp
from jax import lax
from jax.experimental import pallas as pl
from jax.experimental.pallas import tpu as pltpu
from jax.experimental.pallas import tpu_sc as plsc

_LEVELS = 16
_FEATURES = 2
_HASHMAP = 524288
_BASE_RES = 16
_SCALE = 1.3819
_N = 524288
_HIDDEN = 64

_NC, _NS = 2, 16
_NW = _NC * _NS
_CHUNK = _N // _NW
_SUB = 128
_NSUB = _CHUNK // _SUB

_C0 = np.int32(73856093)
_C1 = np.int32(19349663)
_C2 = np.int32(83492791)
_RES = [np.float32(int(_BASE_RES * _SCALE ** l)) for l in range(_LEVELS)]


def _sc_embed_body(x_hbm, tbl_hbm, out_hbm, xv, idx0, idx1, lowv, rows, outv,
                   gsem0, gsem1, osem0, osem1):
    wid = lax.axis_index("c") * _NS + lax.axis_index("s")
    iota = lax.iota(jnp.int32, 16)
    gsems = (gsem0, gsem1)
    osems = (osem0, osem1)

    def prep(g, b):
        pb = wid * _NSUB + g
        pltpu.sync_copy(x_hbm.at[:, pb], xv.at[b])

        @pl.loop(0, _SUB // 16)
        def _hash(gg):
            p0 = gg * 16
            xx = xv[b, 0, pl.ds(p0, 16)]
            yy = xv[b, 1, pl.ds(p0, 16)]
            zz = xv[b, 2, pl.ds(p0, 16)]
            for l in range(_LEVELS):
                px = (xx * _RES[l]).astype(jnp.int32)
                py = (yy * _RES[l]).astype(jnp.int32)
                pz = (zz * _RES[l]).astype(jnp.int32)
                h = (px * _C0) ^ (py * _C1) ^ (pz * _C2)
                slot = h & np.int32(_HASHMAP - 1)
                r0 = (lax.shift_right_logical(slot & np.int32(0x7FF80),
                                              np.int32(2))
                      | (lax.shift_right_logical(slot, np.int32(3))
                         & np.int32(15))
                      | np.int32(l * 131072))
                idx0[b, l, pl.ds(p0, 16)] = r0
                idx1[b, l, pl.ds(p0, 16)] = r0 | np.int32(16)
                lowv[b, l, pl.ds(p0, 16)] = slot & np.int32(7)

        for l in range(_LEVELS):
            pltpu.async_copy(
                tbl_hbm.at[idx0.at[b, l]], rows.at[b, l, 0], gsems[b])
            pltpu.async_copy(
                tbl_hbm.at[idx1.at[b, l]], rows.at[b, l, 1], gsems[b])

    def drain(g, b):
        pb = wid * _NSUB + g
        for l in range(_LEVELS):
            pltpu.make_async_copy(
                tbl_hbm.at[idx0.at[b, l]], rows.at[b, l, 0], gsems[b]).wait()
            pltpu.make_async_copy(
                tbl_hbm.at[idx1.at[b, l]], rows.at[b, l, 1], gsems[b]).wait()

        @pl.when(g >= 2)
        def _():
            pltpu.make_async_copy(
                outv.at[b], out_hbm.at[:, pb], osems[b]).wait()

        @pl.loop(0, _SUB // 16)
        def _transpose(gg):
            p0 = gg * 16
            p_idx = p0 + iota
            for l in range(_LEVELS):
                low = lowv[b, l, pl.ds(p0, 16)]
                lsplat = jnp.full((16,), l, jnp.int32)
                for f in range(_FEATURES):
                    fsplat = jnp.full((16,), f, jnp.int32)
                    vals = plsc.load_gather(
                        rows.at[b], [lsplat, fsplat, p_idx, low])
                    c = 2 * l + f
                    outv[b, c // 8, c % 8, pl.ds(p0, 16)] = vals

        pltpu.async_copy(outv.at[b], out_hbm.at[:, pb], osems[b])

    prep(0, 0)

    @pl.loop(0, _NSUB // 2)
    def _sub(gh):
        g0 = 2 * gh
        prep(g0 + 1, 1)
        drain(g0, 0)

        @pl.when(g0 + 2 < _NSUB)
        def _():
            prep(g0 + 2, 0)

        drain(g0 + 1, 1)

    last = wid * _NSUB + _NSUB - 1
    pltpu.make_async_copy(outv.at[0], out_hbm.at[:, last - 1], osems[0]).wait()
    pltpu.make_async_copy(outv.at[1], out_hbm.at[:, last], osems[1]).wait()


@functools.cache
def _get_sc_embed():
    return pl.kernel(
        _sc_embed_body,
        out_type=jax.ShapeDtypeStruct((4, _N // _SUB, 8, _SUB), jnp.float32),
        mesh=plsc.VectorSubcoreMesh(
            core_axis_name="c", subcore_axis_name="s",
            num_cores=_NC, num_subcores=_NS),
        scratch_types=[
            pltpu.VMEM((2, 3, _SUB), jnp.float32),
            pltpu.VMEM((2, _LEVELS, _SUB), jnp.int32),
            pltpu.VMEM((2, _LEVELS, _SUB), jnp.int32),
            pltpu.VMEM((2, _LEVELS, _SUB), jnp.int32),
            pltpu.VMEM((2, _LEVELS, _FEATURES, _SUB, 8), jnp.float32),
            pltpu.VMEM((2, 4, 8, _SUB), jnp.float32),
            pltpu.SemaphoreType.DMA,
            pltpu.SemaphoreType.DMA,
            pltpu.SemaphoreType.DMA,
            pltpu.SemaphoreType.DMA,
        ],
        compiler_params=pltpu.CompilerParams(
            needs_layout_passes=False, use_tc_tiling_on_sc=False),
    )


_BLK = 4096


def _prep_body(x_ref, xt_ref):
    xt = x_ref[...].T
    for j in range(_BLK // _SUB):
        xt_ref[:, j] = xt[:, j * _SUB:(j + 1) * _SUB]


def _prep(x):
    return pl.pallas_call(
        _prep_body,
        grid=(_N // _BLK,),
        in_specs=[pl.BlockSpec((_BLK, 3), lambda i: (i, 0))],
        out_specs=pl.BlockSpec((3, _BLK // _SUB, _SUB), lambda i: (0, i, 0)),
        out_shape=jax.ShapeDtypeStruct((3, _N // _SUB, _SUB), jnp.float32),
    )(x)


def _mlp_body(ht_ref, vd_ref, w0a_ref, w0b_ref, b0_ref, w1_ref, b1_ref,
              w2_ref, b2_ref, rgbt_ref, sig_ref):
    dg = functools.partial(jax.lax.dot_general,
                           preferred_element_type=jnp.float32)
    z0v = dg(w0b_ref[...], vd_ref[...], (((0,), (1,)), ((), ())))
    z0 = dg(w0a_ref[...], ht_ref[...], (((0,), (0,)), ((), ())))
    h1 = jnp.maximum(z0 + z0v + b0_ref[...], 0.0)
    z1 = dg(w1_ref[...], h1, (((0,), (0,)), ((), ())))
    h2 = jnp.maximum(z1 + b1_ref[...], 0.0)
    ot = dg(w2_ref[...], h2, (((0,), (0,)), ((), ())))
    ot = ot + b2_ref[...]
    rgbt_ref[...] = 1.0 / (1.0 + jnp.exp(-ot[:3, :]))
    sig_ref[...] = jnp.maximum(ot[3:4, :], 0.0)


def _mlp(ht, vd, w0a, w0b, b0, w1, b1, w2, b2):
    nblk = _N // _BLK
    full = lambda i: (0, 0)
    return pl.pallas_call(
        _mlp_body,
        grid=(nblk,),
        in_specs=[
            pl.BlockSpec((2 * _LEVELS, _BLK), lambda i: (0, i)),
            pl.BlockSpec((_BLK, 3), lambda i: (i, 0)),
            pl.BlockSpec((2 * _LEVELS, _HIDDEN), full),
            pl.BlockSpec((3, _HIDDEN), full),
            pl.BlockSpec((_HIDDEN, 1), full),
            pl.BlockSpec((_HIDDEN, _HIDDEN), full),
            pl.BlockSpec((_HIDDEN, 1), full),
            pl.BlockSpec((_HIDDEN, 4), full),
            pl.BlockSpec((4, 1), full),
        ],
        out_specs=[
            pl.BlockSpec((3, _BLK), lambda i: (0, i)),
            pl.BlockSpec((1, _BLK), lambda i: (0, i)),
        ],
        out_shape=[
            jax.ShapeDtypeStruct((3, _N), jnp.float32),
            jax.ShapeDtypeStruct((1, _N), jnp.float32),
        ],
    )(ht, vd, w0a, w0b, b0, w1, b1, w2, b2)


@jax.jit
def kernel(x, view_dir, tables, W0, b0, W1, b1, W2, b2):
    tbl = (tables.reshape(_LEVELS, _HASHMAP // 128, 128, _FEATURES)
           .transpose(0, 1, 3, 2)
           .reshape(_LEVELS * _HASHMAP * _FEATURES // 8, 8))
    xt = _prep(x)
    h4 = _get_sc_embed()(xt, tbl)
    ht = h4.transpose(0, 2, 1, 3).reshape(2 * _LEVELS, _N)
    rgbt, sigt = _mlp(ht, view_dir,
                      W0[:2 * _LEVELS], W0[2 * _LEVELS:], b0.reshape(-1, 1),
                      W1, b1.reshape(-1, 1), W2, b2.reshape(-1, 1))
    return (rgbt.T, sigt.reshape(_N, 1))
